# baseline (speedup 1.0000x reference)
import jax
import jax.numpy as jnp
from jax.experimental import pallas as pl
from jax.experimental.pallas import tpu as pltpu
from jax.experimental.pallas import tpu_sc as plsc

N_NODES = 4096
TOPK = 1024
CHUNK = 512
ROWBLK = 512
NBLK = 1024

SC_NC = 2
SC_NS = 16
SC_NW = SC_NC * SC_NS
SC_BPW = TOPK // SC_NW
SC_GCH = 8


def _rank_topk_kernel(srow_ref, scol_ref, pt_ref, tix_ref, vals_ref):
    srow = srow_ref[:]
    il = jax.lax.broadcasted_iota(jnp.int32, (CHUNK, N_NODES), 0)
    jx = jax.lax.broadcasted_iota(jnp.int32, (CHUNK, N_NODES), 1)
    d = jx - il
    ones_row = jnp.ones((1, CHUNK), jnp.float32)

    def body(c, acc):
        chunk = scol_ref[pl.ds(c * CHUNK, CHUNK), :]
        gt = chunk > srow
        tie = (chunk == srow) & (c * CHUNK < d)
        cnt = (gt | tie).astype(jnp.float32)
        return acc + jax.lax.dot_general(
            ones_row, cnt, (((1,), (0,)), ((), ())),
            preferred_element_type=jnp.float32)

    rank = jax.lax.fori_loop(0, N_NODES // CHUNK, body,
                             jnp.zeros((1, N_NODES), jnp.float32))
    r_col = jax.lax.broadcasted_iota(jnp.int32, (TOPK, 1), 0).astype(jnp.float32)
    mask = rank == r_col
    maskf = mask.astype(jnp.float32)
    pt_ref[:] = maskf.astype(jnp.bfloat16)
    jcol = jax.lax.broadcasted_iota(jnp.int32, (N_NODES, 1), 0).astype(jnp.float32)
    tix_ref[:] = jax.lax.dot_general(
        maskf, jcol, (((1,), (0,)), ((), ())),
        preferred_element_type=jnp.float32).astype(jnp.int32)
    vals_ref[:] = jax.lax.dot_general(
        maskf, scol_ref[:], (((1,), (0,)), ((), ())),
        preferred_element_type=jnp.float32)


def _sc_gather_kernel(g_hbm, h_hbm, idx_hbm, outg_hbm, outh_hbm,
                      idx_v, idx8_v, rowsg_v, rowsh_v, sem_g, sem_h):
    wid = jax.lax.axis_index("s") * SC_NC + jax.lax.axis_index("c")
    base = wid * SC_BPW
    pltpu.sync_copy(idx_hbm.at[pl.ds(base, SC_BPW)], idx_v)
    cp_h = pltpu.async_copy(h_hbm.at[idx_v], rowsh_v, sem_h)
    nch = SC_BPW // SC_GCH
    cps = [None] * nch
    for c in range(nch):
        pltpu.sync_copy(idx_hbm.at[pl.ds(base + c * SC_GCH, SC_GCH)],
                        idx8_v.at[c % 2])
        cps[c] = pltpu.async_copy(g_hbm.at[idx8_v.at[c % 2]],
                                  rowsg_v.at[c % 2], sem_g[c % 2])
        if c > 0:
            cps[c - 1].wait()
            pltpu.sync_copy(rowsg_v.at[(c - 1) % 2],
                            outg_hbm.at[pl.ds(base + (c - 1) * SC_GCH, SC_GCH)])
    cps[nch - 1].wait()
    pltpu.sync_copy(rowsg_v.at[(nch - 1) % 2],
                    outg_hbm.at[pl.ds(base + (nch - 1) * SC_GCH, SC_GCH)])
    cp_h.wait()
    pltpu.sync_copy(rowsh_v, outh_hbm.at[pl.ds(base, SC_BPW)])


def _row_prep_kernel(rowsg_ref, rowsh_ref, tix_ref, vals_ref, r0_ref, nh_ref):
    cols = jax.lax.broadcasted_iota(jnp.int32, (TOPK, N_NODES), 1)
    diag = cols == tix_ref[:]
    r0_ref[:] = ((rowsg_ref[:] != 0.0) | diag).astype(jnp.bfloat16)
    nh_ref[:] = rowsh_ref[:] * vals_ref[:]


def _binarize_kernel(g_ref, a_ref):
    i = pl.program_id(0)
    rows = jax.lax.broadcasted_iota(jnp.int32, (ROWBLK, N_NODES), 0) + i * ROWBLK
    cols = jax.lax.broadcasted_iota(jnp.int32, (ROWBLK, N_NODES), 1)
    gb = g_ref[:]
    a_ref[:] = ((gb != 0.0) | (rows == cols)).astype(jnp.bfloat16)


def _chain_kernel(r0_ref, a_ref, pt_ref, o_ref, x1, x2):
    s = pl.program_id(0)
    nds = pl.ds((s % 4) * NBLK, NBLK)

    def binpat(x, a_blk):
        acc = jax.lax.dot_general(
            x, a_blk, (((1,), (0,)), ((), ())),
            preferred_element_type=jnp.float32)
        return (acc > 0).astype(jnp.bfloat16)

    @pl.when(s < 4)
    def _():
        x1[:, nds] = binpat(r0_ref[:], a_ref[:])

    @pl.when((s >= 4) & (s < 8))
    def _():
        x2[:, nds] = binpat(x1[:], a_ref[:])

    @pl.when((s >= 8) & (s < 12))
    def _():
        x1[:, nds] = binpat(x2[:], a_ref[:])

    @pl.when(s >= 12)
    def _():
        part = jax.lax.dot_general(
            x1[:, nds], pt_ref[:], (((1,), (1,)), ((), ())),
            preferred_element_type=jnp.float32)
        acc = jnp.where(s == 12, 0.0, o_ref[:]) + part

        @pl.when(s < 15)
        def _():
            o_ref[:] = acc

        @pl.when(s == 15)
        def _():
            cb = (acc > 0).astype(jnp.float32)
            deg = jnp.sum(cb, axis=1, keepdims=True)
            deg = jnp.where(deg > 0, deg, 1.0)
            o_ref[:] = cb / deg


def kernel(g, h, Ws, bs):
    n_att = Ws.shape[0]
    in_dim = h.shape[1]
    scores = []
    for i in range(n_att):
        weights = h @ Ws[i] + bs[i]
        scores.append(jax.nn.sigmoid(weights))
    score = jnp.stack(scores, axis=0).sum(axis=0)
    srow = score.reshape(1, N_NODES)
    scol = score.reshape(N_NODES, 1)

    pt, tix, vals = pl.pallas_call(
        _rank_topk_kernel,
        out_shape=(
            jax.ShapeDtypeStruct((TOPK, N_NODES), jnp.bfloat16),
            jax.ShapeDtypeStruct((TOPK, 1), jnp.int32),
            jax.ShapeDtypeStruct((TOPK, 1), jnp.float32),
        ),
    )(srow, scol)

    rows_g, rows_h = pl.kernel(
        _sc_gather_kernel,
        out_type=(
            jax.ShapeDtypeStruct((TOPK, N_NODES), jnp.float32),
            jax.ShapeDtypeStruct((TOPK, in_dim), jnp.float32),
        ),
        mesh=plsc.VectorSubcoreMesh(core_axis_name="c", subcore_axis_name="s"),
        scratch_types=[
            pltpu.VMEM((SC_BPW,), jnp.int32),
            pltpu.VMEM((2, SC_GCH), jnp.int32),
            pltpu.VMEM((2, SC_GCH, N_NODES), jnp.float32),
            pltpu.VMEM((SC_BPW, in_dim), jnp.float32),
            (pltpu.SemaphoreType.DMA, pltpu.SemaphoreType.DMA),
            pltpu.SemaphoreType.DMA,
        ],
    )(g, h, tix.reshape(TOPK))

    a = pl.pallas_call(
        _binarize_kernel,
        grid=(N_NODES // ROWBLK,),
        in_specs=[pl.BlockSpec((ROWBLK, N_NODES), lambda i: (i, 0))],
        out_specs=pl.BlockSpec((ROWBLK, N_NODES), lambda i: (i, 0)),
        out_shape=jax.ShapeDtypeStruct((N_NODES, N_NODES), jnp.bfloat16),
    )(g)

    r0, new_h = pl.pallas_call(
        _row_prep_kernel,
        out_shape=(
            jax.ShapeDtypeStruct((TOPK, N_NODES), jnp.bfloat16),
            jax.ShapeDtypeStruct((TOPK, in_dim), jnp.float32),
        ),
    )(rows_g, rows_h, tix, vals)

    g_out = pl.pallas_call(
        _chain_kernel,
        grid=(16,),
        in_specs=[
            pl.BlockSpec((TOPK, N_NODES), lambda s: (0, 0)),
            pl.BlockSpec((N_NODES, NBLK), lambda s: (0, jnp.minimum(s, 11) % 4)),
            pl.BlockSpec((TOPK, NBLK), lambda s: (0, jnp.maximum(s - 12, 0))),
        ],
        out_specs=pl.BlockSpec((TOPK, TOPK), lambda s: (0, 0)),
        out_shape=jax.ShapeDtypeStruct((TOPK, TOPK), jnp.float32),
        scratch_shapes=[
            pltpu.VMEM((TOPK, N_NODES), jnp.bfloat16),
            pltpu.VMEM((TOPK, N_NODES), jnp.bfloat16),
        ],
        compiler_params=pltpu.CompilerParams(
            dimension_semantics=("arbitrary",)),
    )(r0, a, pt)

    top_idx = tix.reshape(TOPK)
    return (g_out, new_h, top_idx)


# speedup vs baseline: 1.0057x; 1.0057x over previous
import jax
import jax.numpy as jnp
from jax.experimental import pallas as pl
from jax.experimental.pallas import tpu as pltpu
from jax.experimental.pallas import tpu_sc as plsc

N_NODES = 4096
TOPK = 1024
CHUNK = 1024
ROWBLK = 512
NBLK = 1024

SC_NC = 2
SC_NS = 16
SC_NW = SC_NC * SC_NS
SC_BPW = TOPK // SC_NW
SC_GCH = 8


def _rank_topk_kernel(srow_ref, scol_ref, pt_ref, tix_ref, vals_ref):
    srow = srow_ref[:]

    def body(c, acc):
        chunk = scol_ref[pl.ds(c * CHUNK, CHUNK), :]
        gidx = jax.lax.broadcasted_iota(jnp.int32, (CHUNK, N_NODES), 0) + c * CHUNK
        jidx = jax.lax.broadcasted_iota(jnp.int32, (CHUNK, N_NODES), 1)
        gt = chunk > srow
        eqlt = (chunk == srow) & (gidx < jidx)
        cnt = jnp.where(gt | eqlt, 1, 0)
        return acc + jnp.sum(cnt, axis=0, keepdims=True)

    rank = jax.lax.fori_loop(0, N_NODES // CHUNK, body,
                             jnp.zeros((1, N_NODES), jnp.int32))
    r_col = jax.lax.broadcasted_iota(jnp.int32, (TOPK, 1), 0)
    mask = rank == r_col
    pt_ref[:] = mask.astype(jnp.bfloat16)
    jidx2 = jax.lax.broadcasted_iota(jnp.int32, (TOPK, N_NODES), 1)
    tix_ref[:] = jnp.sum(jnp.where(mask, jidx2, 0), axis=1, keepdims=True)
    vals_ref[:] = jnp.sum(
        jnp.where(mask, jnp.broadcast_to(srow, (TOPK, N_NODES)), 0.0),
        axis=1, keepdims=True)


def _sc_gather_kernel(g_hbm, h_hbm, idx_hbm, outg_hbm, outh_hbm,
                      idx_v, idx8_v, rowsg_v, rowsh_v, sem_g, sem_h):
    wid = jax.lax.axis_index("s") * SC_NC + jax.lax.axis_index("c")
    base = wid * SC_BPW
    pltpu.sync_copy(idx_hbm.at[pl.ds(base, SC_BPW)], idx_v)
    cp_h = pltpu.async_copy(h_hbm.at[idx_v], rowsh_v, sem_h)
    nch = SC_BPW // SC_GCH
    cps = [None] * nch
    for c in range(nch):
        pltpu.sync_copy(idx_hbm.at[pl.ds(base + c * SC_GCH, SC_GCH)],
                        idx8_v.at[c % 2])
        cps[c] = pltpu.async_copy(g_hbm.at[idx8_v.at[c % 2]],
                                  rowsg_v.at[c % 2], sem_g[c % 2])
        if c > 0:
            cps[c - 1].wait()
            pltpu.sync_copy(rowsg_v.at[(c - 1) % 2],
                            outg_hbm.at[pl.ds(base + (c - 1) * SC_GCH, SC_GCH)])
    cps[nch - 1].wait()
    pltpu.sync_copy(rowsg_v.at[(nch - 1) % 2],
                    outg_hbm.at[pl.ds(base + (nch - 1) * SC_GCH, SC_GCH)])
    cp_h.wait()
    pltpu.sync_copy(rowsh_v, outh_hbm.at[pl.ds(base, SC_BPW)])


def _row_prep_kernel(rowsg_ref, rowsh_ref, tix_ref, vals_ref, r0_ref, nh_ref):
    cols = jax.lax.broadcasted_iota(jnp.int32, (TOPK, N_NODES), 1)
    diag = cols == tix_ref[:]
    r0_ref[:] = ((rowsg_ref[:] != 0.0) | diag).astype(jnp.bfloat16)
    nh_ref[:] = rowsh_ref[:] * vals_ref[:]


def _binarize_kernel(g_ref, a_ref):
    i = pl.program_id(0)
    rows = jax.lax.broadcasted_iota(jnp.int32, (ROWBLK, N_NODES), 0) + i * ROWBLK
    cols = jax.lax.broadcasted_iota(jnp.int32, (ROWBLK, N_NODES), 1)
    gb = g_ref[:]
    a_ref[:] = ((gb != 0.0) | (rows == cols)).astype(jnp.bfloat16)


def _chain_kernel(r0_ref, a_ref, pt_ref, o_ref, x1, x2):
    s = pl.program_id(0)
    nds = pl.ds((s % 4) * NBLK, NBLK)

    def binpat(x, a_blk):
        acc = jax.lax.dot_general(
            x, a_blk, (((1,), (0,)), ((), ())),
            preferred_element_type=jnp.float32)
        return (acc > 0).astype(jnp.bfloat16)

    @pl.when(s < 4)
    def _():
        x1[:, nds] = binpat(r0_ref[:], a_ref[:])

    @pl.when((s >= 4) & (s < 8))
    def _():
        x2[:, nds] = binpat(x1[:], a_ref[:])

    @pl.when((s >= 8) & (s < 12))
    def _():
        x1[:, nds] = binpat(x2[:], a_ref[:])

    @pl.when(s >= 12)
    def _():
        part = jax.lax.dot_general(
            x1[:, nds], pt_ref[:], (((1,), (1,)), ((), ())),
            preferred_element_type=jnp.float32)
        acc = jnp.where(s == 12, 0.0, o_ref[:]) + part

        @pl.when(s < 15)
        def _():
            o_ref[:] = acc

        @pl.when(s == 15)
        def _():
            cb = (acc > 0).astype(jnp.float32)
            deg = jnp.sum(cb, axis=1, keepdims=True)
            deg = jnp.where(deg > 0, deg, 1.0)
            o_ref[:] = cb / deg


def kernel(g, h, Ws, bs):
    n_att = Ws.shape[0]
    in_dim = h.shape[1]
    scores = []
    for i in range(n_att):
        weights = h @ Ws[i] + bs[i]
        scores.append(jax.nn.sigmoid(weights))
    score = jnp.stack(scores, axis=0).sum(axis=0)
    srow = score.reshape(1, N_NODES)
    scol = score.reshape(N_NODES, 1)

    pt, tix, vals = pl.pallas_call(
        _rank_topk_kernel,
        out_shape=(
            jax.ShapeDtypeStruct((TOPK, N_NODES), jnp.bfloat16),
            jax.ShapeDtypeStruct((TOPK, 1), jnp.int32),
            jax.ShapeDtypeStruct((TOPK, 1), jnp.float32),
        ),
    )(srow, scol)

    rows_g, rows_h = pl.kernel(
        _sc_gather_kernel,
        out_type=(
            jax.ShapeDtypeStruct((TOPK, N_NODES), jnp.float32),
            jax.ShapeDtypeStruct((TOPK, in_dim), jnp.float32),
        ),
        mesh=plsc.VectorSubcoreMesh(core_axis_name="c", subcore_axis_name="s"),
        scratch_types=[
            pltpu.VMEM((SC_BPW,), jnp.int32),
            pltpu.VMEM((2, SC_GCH), jnp.int32),
            pltpu.VMEM((2, SC_GCH, N_NODES), jnp.float32),
            pltpu.VMEM((SC_BPW, in_dim), jnp.float32),
            (pltpu.SemaphoreType.DMA, pltpu.SemaphoreType.DMA),
            pltpu.SemaphoreType.DMA,
        ],
    )(g, h, tix.reshape(TOPK))

    a = pl.pallas_call(
        _binarize_kernel,
        grid=(N_NODES // ROWBLK,),
        in_specs=[pl.BlockSpec((ROWBLK, N_NODES), lambda i: (i, 0))],
        out_specs=pl.BlockSpec((ROWBLK, N_NODES), lambda i: (i, 0)),
        out_shape=jax.ShapeDtypeStruct((N_NODES, N_NODES), jnp.bfloat16),
    )(g)

    r0, new_h = pl.pallas_call(
        _row_prep_kernel,
        out_shape=(
            jax.ShapeDtypeStruct((TOPK, N_NODES), jnp.bfloat16),
            jax.ShapeDtypeStruct((TOPK, in_dim), jnp.float32),
        ),
    )(rows_g, rows_h, tix, vals)

    g_out = pl.pallas_call(
        _chain_kernel,
        grid=(16,),
        in_specs=[
            pl.BlockSpec((TOPK, N_NODES), lambda s: (0, 0)),
            pl.BlockSpec((N_NODES, NBLK), lambda s: (0, jnp.minimum(s, 11) % 4)),
            pl.BlockSpec((TOPK, NBLK), lambda s: (0, jnp.maximum(s - 12, 0))),
        ],
        out_specs=pl.BlockSpec((TOPK, TOPK), lambda s: (0, 0)),
        out_shape=jax.ShapeDtypeStruct((TOPK, TOPK), jnp.float32),
        scratch_shapes=[
            pltpu.VMEM((TOPK, N_NODES), jnp.bfloat16),
            pltpu.VMEM((TOPK, N_NODES), jnp.bfloat16),
        ],
        compiler_params=pltpu.CompilerParams(
            dimension_semantics=("arbitrary",)),
    )(r0, a, pt)

    top_idx = tix.reshape(TOPK)
    return (g_out, new_h, top_idx)
